# baseline (speedup 1.0000x reference)
import jax
import jax.numpy as jnp
from jax.experimental import pallas as pl
from jax.experimental.pallas import tpu as pltpu

IN_DIM = 128
OUT_DIM = 256
MEM_LEN = 100000
BETA = 2000.0

BLK = 5000
NBLK = MEM_LEN // BLK


def _body(x_ref, mean_ref, std_ref, w_ref, b_ref, mem_ref, md_any,
          loss_ref, enc_out_ref, out_mem_ref, out_md_any,
          enc_ref, min_ref, md_sem):
    i = pl.program_id(0)

    @pl.when(i == 0)
    def _encode():
        pltpu.make_async_copy(md_any, out_md_any, md_sem).start()
        xv = x_ref[...]
        std = std_ref[...]
        new = jnp.where(std == 0.0, 0.0, (xv - mean_ref[...]) / std)
        logits = jnp.dot(new, w_ref[...],
                         preferred_element_type=jnp.float32) + b_ref[...]
        m = jnp.max(logits)
        lse = jnp.log(jnp.sum(jnp.exp(logits - m))) + m
        enc = logits - lse
        enc_ref[...] = enc
        enc_out_ref[...] = enc
        min_ref[0] = jnp.inf

    blk = mem_ref[...]
    out_mem_ref[...] = blk
    d = jnp.sum(jnp.abs(blk - enc_ref[...]), axis=1)
    min_ref[0] = jnp.minimum(min_ref[0], jnp.min(d))

    @pl.when(i == NBLK - 1)
    def _emit_loss():
        loss_ref[...] = jnp.full((1, 1), min_ref[0], jnp.float32)
        pltpu.make_async_copy(md_any, out_md_any, md_sem).wait()


def _fixup_body(mem_any, md_any, loss_ref, enc_ref, x_ref, pos_ref,
                out_mem_any, out_md_any, enc_vmem, x_vmem, sem_a, sem_b):
    @pl.when(loss_ref[0] <= BETA)
    def _():
        enc_vmem[...] = enc_ref[...]
        x_vmem[...] = x_ref[...]
        c1 = pltpu.async_copy(
            enc_vmem, out_mem_any.at[pl.ds(pos_ref[0], 1), :], sem_a)
        c2 = pltpu.async_copy(
            x_vmem, out_md_any.at[pl.ds(pos_ref[0], 1), :], sem_b)
        c1.wait()
        c2.wait()


def _fixup(mem_copied, md_copied, loss1, enc, x, pos):
    return pl.pallas_call(
        _fixup_body,
        grid=(),
        in_specs=[
            pl.BlockSpec(memory_space=pltpu.MemorySpace.HBM),
            pl.BlockSpec(memory_space=pltpu.MemorySpace.HBM),
            pl.BlockSpec(memory_space=pltpu.SMEM),
            pl.BlockSpec(memory_space=pltpu.VMEM),
            pl.BlockSpec(memory_space=pltpu.VMEM),
            pl.BlockSpec(memory_space=pltpu.SMEM),
        ],
        out_specs=[
            pl.BlockSpec(memory_space=pltpu.MemorySpace.HBM),
            pl.BlockSpec(memory_space=pltpu.MemorySpace.HBM),
        ],
        out_shape=[
            jax.ShapeDtypeStruct((MEM_LEN, OUT_DIM), jnp.float32),
            jax.ShapeDtypeStruct((MEM_LEN, IN_DIM), jnp.float32),
        ],
        scratch_shapes=[
            pltpu.VMEM((1, OUT_DIM), jnp.float32),
            pltpu.VMEM((1, IN_DIM), jnp.float32),
            pltpu.SemaphoreType.DMA,
            pltpu.SemaphoreType.DMA,
        ],
        input_output_aliases={0: 0, 1: 1},
    )(mem_copied, md_copied, loss1, enc, x, pos)


def kernel(x, mean, std, W_enc, b_enc, memory, mem_data, count):
    pos = jnp.asarray(count % MEM_LEN, jnp.int32).reshape(1)
    mean2 = mean.reshape(1, IN_DIM)
    std2 = std.reshape(1, IN_DIM)
    b2 = b_enc.reshape(1, OUT_DIM)

    def big_map(i):
        return (i, 0)

    def const_map(i):
        return (0, 0)

    loss2d, enc, mem_copied, md_copied = pl.pallas_call(
        _body,
        grid=(NBLK,),
        in_specs=[
            pl.BlockSpec((1, IN_DIM), const_map),
            pl.BlockSpec((1, IN_DIM), const_map),
            pl.BlockSpec((1, IN_DIM), const_map),
            pl.BlockSpec((IN_DIM, OUT_DIM), const_map),
            pl.BlockSpec((1, OUT_DIM), const_map),
            pl.BlockSpec((BLK, OUT_DIM), big_map),
            pl.BlockSpec(memory_space=pltpu.MemorySpace.HBM),
        ],
        out_specs=[
            pl.BlockSpec((1, 1), const_map),
            pl.BlockSpec((1, OUT_DIM), const_map),
            pl.BlockSpec((BLK, OUT_DIM), big_map),
            pl.BlockSpec(memory_space=pltpu.MemorySpace.HBM),
        ],
        out_shape=[
            jax.ShapeDtypeStruct((1, 1), jnp.float32),
            jax.ShapeDtypeStruct((1, OUT_DIM), jnp.float32),
            jax.ShapeDtypeStruct((MEM_LEN, OUT_DIM), jnp.float32),
            jax.ShapeDtypeStruct((MEM_LEN, IN_DIM), jnp.float32),
        ],
        scratch_shapes=[
            pltpu.VMEM((1, OUT_DIM), jnp.float32),
            pltpu.SMEM((1,), jnp.float32),
            pltpu.SemaphoreType.DMA,
        ],
        compiler_params=pltpu.CompilerParams(
            dimension_semantics=("arbitrary",),
        ),
    )(x, mean2, std2, W_enc, b2, memory, mem_data)

    loss1 = loss2d.reshape(1)
    new_memory, new_mem_data = _fixup(mem_copied, md_copied, loss1, enc, x, pos)

    return loss2d.reshape(()), new_memory, new_mem_data


# speedup vs baseline: 15.7506x; 15.7506x over previous
import jax
import jax.numpy as jnp
from jax.experimental import pallas as pl
from jax.experimental.pallas import tpu as pltpu

IN_DIM = 128
OUT_DIM = 256
MEM_LEN = 100000
BETA = 2000.0

BLK = 4000
NBLK = MEM_LEN // BLK


def _body(x_ref, mean_ref, std_ref, w_ref, b_ref, mem_ref, md_ref,
          loss_ref, enc_out_ref, out_mem_ref, out_md_ref,
          enc_ref, min_ref):
    i = pl.program_id(0)

    @pl.when(i == 0)
    def _encode():
        xv = x_ref[...]
        std = std_ref[...]
        new = jnp.where(std == 0.0, 0.0, (xv - mean_ref[...]) / std)
        logits = jnp.dot(new, w_ref[...],
                         preferred_element_type=jnp.float32) + b_ref[...]
        m = jnp.max(logits)
        lse = jnp.log(jnp.sum(jnp.exp(logits - m))) + m
        enc = logits - lse
        enc_ref[...] = enc
        enc_out_ref[...] = enc
        min_ref[0] = jnp.inf

    blk = mem_ref[...]
    out_mem_ref[...] = blk
    out_md_ref[...] = md_ref[...]
    d = jnp.sum(jnp.abs(blk - enc_ref[...]), axis=1)
    min_ref[0] = jnp.minimum(min_ref[0], jnp.min(d))

    @pl.when(i == NBLK - 1)
    def _emit_loss():
        loss_ref[...] = jnp.full((1, 1), min_ref[0], jnp.float32)


def _fixup_body(mem_any, md_any, loss_ref, enc_ref, x_ref, pos_ref,
                out_mem_any, out_md_any, enc_vmem, x_vmem, sem_a, sem_b):
    @pl.when(loss_ref[0] <= BETA)
    def _():
        enc_vmem[...] = enc_ref[...]
        x_vmem[...] = x_ref[...]
        c1 = pltpu.async_copy(
            enc_vmem, out_mem_any.at[pl.ds(pos_ref[0], 1), :], sem_a)
        c2 = pltpu.async_copy(
            x_vmem, out_md_any.at[pl.ds(pos_ref[0], 1), :], sem_b)
        c1.wait()
        c2.wait()


def _fixup(mem_copied, md_copied, loss1, enc, x, pos):
    return pl.pallas_call(
        _fixup_body,
        grid=(),
        in_specs=[
            pl.BlockSpec(memory_space=pltpu.MemorySpace.HBM),
            pl.BlockSpec(memory_space=pltpu.MemorySpace.HBM),
            pl.BlockSpec(memory_space=pltpu.SMEM),
            pl.BlockSpec(memory_space=pltpu.VMEM),
            pl.BlockSpec(memory_space=pltpu.VMEM),
            pl.BlockSpec(memory_space=pltpu.SMEM),
        ],
        out_specs=[
            pl.BlockSpec(memory_space=pltpu.MemorySpace.HBM),
            pl.BlockSpec(memory_space=pltpu.MemorySpace.HBM),
        ],
        out_shape=[
            jax.ShapeDtypeStruct((MEM_LEN, OUT_DIM), jnp.float32),
            jax.ShapeDtypeStruct((MEM_LEN, IN_DIM), jnp.float32),
        ],
        scratch_shapes=[
            pltpu.VMEM((1, OUT_DIM), jnp.float32),
            pltpu.VMEM((1, IN_DIM), jnp.float32),
            pltpu.SemaphoreType.DMA,
            pltpu.SemaphoreType.DMA,
        ],
        input_output_aliases={0: 0, 1: 1},
    )(mem_copied, md_copied, loss1, enc, x, pos)


def kernel(x, mean, std, W_enc, b_enc, memory, mem_data, count):
    pos = jnp.asarray(count % MEM_LEN, jnp.int32).reshape(1)
    mean2 = mean.reshape(1, IN_DIM)
    std2 = std.reshape(1, IN_DIM)
    b2 = b_enc.reshape(1, OUT_DIM)

    def big_map(i):
        return (i, 0)

    def const_map(i):
        return (0, 0)

    loss2d, enc, mem_copied, md_copied = pl.pallas_call(
        _body,
        grid=(NBLK,),
        in_specs=[
            pl.BlockSpec((1, IN_DIM), const_map),
            pl.BlockSpec((1, IN_DIM), const_map),
            pl.BlockSpec((1, IN_DIM), const_map),
            pl.BlockSpec((IN_DIM, OUT_DIM), const_map),
            pl.BlockSpec((1, OUT_DIM), const_map),
            pl.BlockSpec((BLK, OUT_DIM), big_map),
            pl.BlockSpec((BLK, IN_DIM), big_map),
        ],
        out_specs=[
            pl.BlockSpec((1, 1), const_map),
            pl.BlockSpec((1, OUT_DIM), const_map),
            pl.BlockSpec((BLK, OUT_DIM), big_map),
            pl.BlockSpec((BLK, IN_DIM), big_map),
        ],
        out_shape=[
            jax.ShapeDtypeStruct((1, 1), jnp.float32),
            jax.ShapeDtypeStruct((1, OUT_DIM), jnp.float32),
            jax.ShapeDtypeStruct((MEM_LEN, OUT_DIM), jnp.float32),
            jax.ShapeDtypeStruct((MEM_LEN, IN_DIM), jnp.float32),
        ],
        scratch_shapes=[
            pltpu.VMEM((1, OUT_DIM), jnp.float32),
            pltpu.SMEM((1,), jnp.float32),
        ],
        compiler_params=pltpu.CompilerParams(
            dimension_semantics=("arbitrary",),
        ),
    )(x, mean2, std2, W_enc, b2, memory, mem_data)

    loss1 = loss2d.reshape(1)
    new_memory, new_mem_data = _fixup(mem_copied, md_copied, loss1, enc, x, pos)

    return loss2d.reshape(()), new_memory, new_mem_data
